# baseline (speedup 1.0000x reference)
import functools

import jax
import jax.numpy as jnp
from jax import lax
from jax.experimental import pallas as pl
from jax.experimental.pallas import tpu as pltpu
from jax.experimental.pallas import tpu_sc as plsc

N_NODES = 10000
N_EDGES = 320000
D_FEAT = 128
D_HALF = 64
CH = 128
NS = 16
NC = 2
GB = 8
NG = 20
NCHUNK = NG * GB
PER_TILE = NCHUNK * CH
E_PAD = NS * PER_TILE
EP_ROWS = E_PAD // CH
ACC_R = 20480
RZ = ACC_R // NS
DUMMY = 2 * N_NODES
K0G = NG // 2


NBUF = 5


def _sc_layer_body(t_hbm, src_hbm, dstx_hbm, z64_hbm, out_hbm,
                   is2, id2, rows0, rows1, rows2, rows3, rows4, acc,
                   sg0, sg1, sg2, sg3, sg4, ss0, ss1, ss2, ss3, ss4):
    rows = (rows0, rows1, rows2, rows3, rows4)
    sem_g = (sg0, sg1, sg2, sg3, sg4)
    sem_s = (ss0, ss1, ss2, ss3, ss4)

    c = lax.axis_index("c")
    s = lax.axis_index("s")

    pltpu.sync_copy(z64_hbm, rows0)
    r0 = s * RZ
    for j in range(RZ // CH):
        pltpu.sync_copy(rows0, acc.at[pl.ds(r0 + j * CH, CH)])
    plsc.subcore_barrier()

    row_base = s * NCHUNK

    @pl.loop(0, NG)
    def _grp(g):
        rb = row_base + g * GB
        pltpu.sync_copy(src_hbm.at[c, pl.ds(rb, GB)], is2)
        pltpu.sync_copy(dstx_hbm.at[pl.ds(rb, GB)], id2)
        depth = NBUF - 2
        gd = [None] * NBUF
        sd = [None] * NBUF

        def consume(jj):
            bb = jj % NBUF
            gd[bb].wait()
            sd[bb] = pltpu.async_copy(rows[bb], acc.at[id2.at[jj]],
                                      sem_s[bb], add=True)

        for j in range(GB):
            b = j % NBUF
            if sd[b] is not None:
                sd[b].wait()
                sd[b] = None
            gd[b] = pltpu.async_copy(t_hbm.at[is2.at[j]], rows[b], sem_g[b])
            if j >= depth:
                consume(j - depth)
        for jj in range(GB - depth, GB):
            consume(jj)
        for d in sd:
            if d is not None:
                d.wait()

    plsc.subcore_barrier()

    for j in range(RZ // CH):
        sl = pl.ds(r0 + j * CH, CH)
        pltpu.sync_copy(acc.at[sl], rows0)
        pltpu.sync_copy(rows0, out_hbm.at[c, sl])


DEG_ROWS_PER_CORE = EP_ROWS // NC
DEG_RPT = DEG_ROWS_PER_CORE // NS
DEG_NG = DEG_RPT // GB


def _sc_deg_body(dstx_hbm, z16_hbm, ones16_hbm, dg_hbm,
                 id2, degt, onesb, dga):
    c = lax.axis_index("c")
    s = lax.axis_index("s")

    pltpu.sync_copy(z16_hbm, degt)
    pltpu.sync_copy(ones16_hbm, onesb)
    r0 = s * RZ
    for j in range(RZ // CH):
        pltpu.sync_copy(degt, dga.at[pl.ds(r0 + j * CH, CH)])
    plsc.subcore_barrier()

    row_base = c * DEG_ROWS_PER_CORE + s * DEG_RPT

    @pl.loop(0, DEG_NG)
    def _grp(g):
        pltpu.sync_copy(dstx_hbm.at[pl.ds(row_base + g * GB, GB)], id2)
        for j in range(GB):
            pltpu.sync_copy(onesb, dga.at[id2.at[j]], add=True)

    plsc.subcore_barrier()

    for j in range(RZ // CH):
        sl = pl.ds(r0 + j * CH, CH)
        pltpu.sync_copy(dga.at[sl], degt)
        pltpu.sync_copy(degt, dg_hbm.at[c, sl])


@functools.lru_cache(maxsize=None)
def _make_sc_layer():
    mesh = plsc.VectorSubcoreMesh(core_axis_name="c", subcore_axis_name="s")
    out_type = [jax.ShapeDtypeStruct((NC, ACC_R, D_HALF), jnp.float32)]
    scratch = [
        pltpu.VMEM((GB, CH), jnp.int32),
        pltpu.VMEM((GB, CH), jnp.int32),
    ] + [pltpu.VMEM((CH, D_HALF), jnp.float32)] * NBUF + [
        pltpu.VMEM_SHARED((ACC_R, D_HALF), jnp.float32),
    ] + [pltpu.SemaphoreType.DMA] * (2 * NBUF)
    return pl.kernel(
        _sc_layer_body,
        out_type=out_type,
        mesh=mesh,
        scratch_types=scratch,
        compiler_params=pltpu.CompilerParams(use_tc_tiling_on_sc=False),
        name="sc_curv_aggregate",
    )


@functools.lru_cache(maxsize=None)
def _make_sc_deg():
    mesh = plsc.VectorSubcoreMesh(core_axis_name="c", subcore_axis_name="s")
    out_type = [jax.ShapeDtypeStruct((NC, ACC_R, 16), jnp.float32)]
    scratch = [
        pltpu.VMEM((GB, CH), jnp.int32),
        pltpu.VMEM((CH, 16), jnp.float32),
        pltpu.VMEM((CH, 16), jnp.float32),
        pltpu.VMEM_SHARED((ACC_R, 16), jnp.float32),
    ]
    return pl.kernel(
        _sc_deg_body,
        out_type=out_type,
        mesh=mesh,
        scratch_types=scratch,
        compiler_params=pltpu.CompilerParams(use_tc_tiling_on_sc=False),
        name="sc_curv_degree",
    )


R_BLK = 2000
N_BLKS = N_NODES // R_BLK


def _tc_mix_body(a0p, a1p, a0q, a1q, dgp, dgq, Wp, Wn, bp, bn, wv, out):
    degp = jnp.maximum(dgp[:, 0:1], 1.0)
    degn = jnp.maximum(dgq[:, 0:1], 1.0)
    dot = functools.partial(jnp.dot, preferred_element_type=jnp.float32)
    hp = (dot(a0p[...] / degp, Wp[0:D_HALF, :])
          + dot(a1p[...] / degp, Wp[D_HALF:D_FEAT, :]) + bp[0:1, :])
    hn = (dot(a0q[...] / degn, Wn[0:D_HALF, :])
          + dot(a1q[...] / degn, Wn[D_HALF:D_FEAT, :]) + bn[0:1, :])
    out[...] = wv[0, 0] * hp + wv[0, 1] * hn


def _tc_bn_body(h_ref, gamma, beta, ht):
    n = N_NODES
    h = h_ref[...]
    mean = jnp.mean(h, axis=0, keepdims=True)
    var = jnp.mean((h - mean) * (h - mean), axis=0, keepdims=True)
    h = (h - mean) * lax.rsqrt(var + 1e-5) * gamma[0:1, :] + beta[0:1, :]
    h = jnp.maximum(h, 0.0)
    ht[0, :, :] = h[:, 0:D_HALF]
    ht[1, :, :] = h[:, D_HALF:D_FEAT]


def _mk_mix():
    pblk = lambda: pl.BlockSpec((R_BLK, D_HALF), lambda k: (k, 0))
    qblk = lambda: pl.BlockSpec((R_BLK, D_HALF), lambda k: (k + N_BLKS, 0))
    wblk = lambda r, c: pl.BlockSpec((r, c), lambda k: (0, 0))
    return pl.pallas_call(
        _tc_mix_body,
        grid=(N_BLKS,),
        in_specs=[
            pblk(), pblk(), qblk(), qblk(),
            pl.BlockSpec((R_BLK, 16), lambda k: (k, 0)),
            pl.BlockSpec((R_BLK, 16), lambda k: (k + N_BLKS, 0)),
            wblk(D_FEAT, D_FEAT), wblk(D_FEAT, D_FEAT),
            wblk(1, D_FEAT), wblk(1, D_FEAT), wblk(1, 2),
        ],
        out_specs=pl.BlockSpec((R_BLK, D_FEAT), lambda k: (k, 0)),
        out_shape=jax.ShapeDtypeStruct((N_NODES, D_FEAT), jnp.float32),
    )


_tc_mix = _mk_mix()

_tc_bn = pl.pallas_call(
    _tc_bn_body,
    out_shape=jax.ShapeDtypeStruct((2, N_NODES, D_HALF), jnp.float32),
)


def kernel(x, edge_index, edge_curvature, edge_attr, W_pos0, b_pos0, W_neg0,
           b_neg0, W_pos1, b_pos1, W_neg1, b_neg1, cw0, cw1, bn_gamma,
           bn_beta):
    n = N_NODES
    src = edge_index[0].astype(jnp.int32)
    dst = edge_index[1].astype(jnp.int32)
    pos = edge_curvature > 0
    neg = edge_curvature < 0
    dstx = jnp.where(pos, dst, jnp.where(neg, dst + n, DUMMY))
    pad = E_PAD - N_EDGES
    src_p = jnp.concatenate([src, jnp.zeros((pad,), jnp.int32)])
    dstx_p = jnp.concatenate([dstx, jnp.full((pad,), DUMMY, jnp.int32)])
    src2 = jnp.stack([src_p, src_p + n]).reshape(NC, EP_ROWS, CH)
    dstx2 = dstx_p.reshape(EP_ROWS, CH)

    t0 = jnp.concatenate([x[:, :D_HALF], x[:, D_HALF:]], axis=0)

    z64 = jnp.zeros((CH, D_HALF), jnp.float32)
    z16 = jnp.zeros((CH, 16), jnp.float32)
    ones16 = z16.at[:, 0].set(1.0)

    (dg,) = _make_sc_deg()(dstx2, z16, ones16)
    (acc,) = _make_sc_layer()(t0, src2, dstx2, z64)
    dgs = dg[0] + dg[1]

    w0 = jax.nn.softmax(cw0).reshape(1, 2)
    w1 = jax.nn.softmax(cw1).reshape(1, 2)
    r1 = lambda v: v.reshape(1, -1)

    hpre = _tc_mix(acc[0], acc[1], acc[0], acc[1], dgs, dgs, W_pos0, W_neg0,
                   r1(b_pos0), r1(b_neg0), w0)
    ht = _tc_bn(hpre, r1(bn_gamma), r1(bn_beta)).reshape(2 * n, D_HALF)

    (acc1,) = _make_sc_layer()(ht, src2, dstx2, z64)

    out = _tc_mix(acc1[0], acc1[1], acc1[0], acc1[1], dgs, dgs, W_pos1,
                  W_neg1, r1(b_pos1), r1(b_neg1), w1)
    return out


# speedup vs baseline: 1.0045x; 1.0045x over previous
import functools

import jax
import jax.numpy as jnp
from jax import lax
from jax.experimental import pallas as pl
from jax.experimental.pallas import tpu as pltpu
from jax.experimental.pallas import tpu_sc as plsc

N_NODES = 10000
N_EDGES = 320000
D_FEAT = 128
D_HALF = 64
CH = 128
NS = 16
NC = 2
GB = 8
NG = 20
NCHUNK = NG * GB
PER_TILE = NCHUNK * CH
E_PAD = NS * PER_TILE
EP_ROWS = E_PAD // CH
ACC_R = 20480
RZ = ACC_R // NS
DUMMY = 2 * N_NODES
K0G = NG // 2


NBUF = 4


def _sc_layer_body(t_hbm, src_hbm, dstx_hbm, z64_hbm, out_hbm,
                   is2, id2, rows0, rows1, rows2, rows3, acc,
                   sg0, sg1, sg2, sg3, ss0, ss1, ss2, ss3):
    rows = (rows0, rows1, rows2, rows3)
    sem_g = (sg0, sg1, sg2, sg3)
    sem_s = (ss0, ss1, ss2, ss3)

    c = lax.axis_index("c")
    s = lax.axis_index("s")

    pltpu.sync_copy(z64_hbm, rows0)
    r0 = s * RZ
    for j in range(RZ // CH):
        pltpu.sync_copy(rows0, acc.at[pl.ds(r0 + j * CH, CH)])
    plsc.subcore_barrier()

    row_base = s * NCHUNK

    @pl.loop(0, NG)
    def _grp(g):
        rb = row_base + g * GB
        pltpu.sync_copy(src_hbm.at[c, pl.ds(rb, GB)], is2)
        pltpu.sync_copy(dstx_hbm.at[pl.ds(rb, GB)], id2)
        depth = NBUF - 2
        gd = [None] * NBUF
        sd = [None] * NBUF

        def consume(jj):
            bb = jj % NBUF
            gd[bb].wait()
            sd[bb] = pltpu.async_copy(rows[bb], acc.at[id2.at[jj]],
                                      sem_s[bb], add=True)

        for j in range(GB):
            b = j % NBUF
            if sd[b] is not None:
                sd[b].wait()
                sd[b] = None
            gd[b] = pltpu.async_copy(t_hbm.at[is2.at[j]], rows[b], sem_g[b])
            if j >= depth:
                consume(j - depth)
        for jj in range(GB - depth, GB):
            consume(jj)
        for d in sd:
            if d is not None:
                d.wait()

    plsc.subcore_barrier()

    for j in range(RZ // CH):
        sl = pl.ds(r0 + j * CH, CH)
        pltpu.sync_copy(acc.at[sl], rows0)
        pltpu.sync_copy(rows0, out_hbm.at[c, sl])


DEG_ROWS_PER_CORE = EP_ROWS // NC
DEG_RPT = DEG_ROWS_PER_CORE // NS
DEG_NG = DEG_RPT // GB


def _sc_deg_body(dstx_hbm, z16_hbm, ones16_hbm, dg_hbm,
                 id2, degt, onesb, dga):
    c = lax.axis_index("c")
    s = lax.axis_index("s")

    pltpu.sync_copy(z16_hbm, degt)
    pltpu.sync_copy(ones16_hbm, onesb)
    r0 = s * RZ
    for j in range(RZ // CH):
        pltpu.sync_copy(degt, dga.at[pl.ds(r0 + j * CH, CH)])
    plsc.subcore_barrier()

    row_base = c * DEG_ROWS_PER_CORE + s * DEG_RPT

    @pl.loop(0, DEG_NG)
    def _grp(g):
        pltpu.sync_copy(dstx_hbm.at[pl.ds(row_base + g * GB, GB)], id2)
        for j in range(GB):
            pltpu.sync_copy(onesb, dga.at[id2.at[j]], add=True)

    plsc.subcore_barrier()

    for j in range(RZ // CH):
        sl = pl.ds(r0 + j * CH, CH)
        pltpu.sync_copy(dga.at[sl], degt)
        pltpu.sync_copy(degt, dg_hbm.at[c, sl])


@functools.lru_cache(maxsize=None)
def _make_sc_layer():
    mesh = plsc.VectorSubcoreMesh(core_axis_name="c", subcore_axis_name="s")
    out_type = [jax.ShapeDtypeStruct((NC, ACC_R, D_HALF), jnp.float32)]
    scratch = [
        pltpu.VMEM((GB, CH), jnp.int32),
        pltpu.VMEM((GB, CH), jnp.int32),
    ] + [pltpu.VMEM((CH, D_HALF), jnp.float32)] * NBUF + [
        pltpu.VMEM_SHARED((ACC_R, D_HALF), jnp.float32),
    ] + [pltpu.SemaphoreType.DMA] * (2 * NBUF)
    return pl.kernel(
        _sc_layer_body,
        out_type=out_type,
        mesh=mesh,
        scratch_types=scratch,
        compiler_params=pltpu.CompilerParams(use_tc_tiling_on_sc=False),
        name="sc_curv_aggregate",
    )


@functools.lru_cache(maxsize=None)
def _make_sc_deg():
    mesh = plsc.VectorSubcoreMesh(core_axis_name="c", subcore_axis_name="s")
    out_type = [jax.ShapeDtypeStruct((NC, ACC_R, 16), jnp.float32)]
    scratch = [
        pltpu.VMEM((GB, CH), jnp.int32),
        pltpu.VMEM((CH, 16), jnp.float32),
        pltpu.VMEM((CH, 16), jnp.float32),
        pltpu.VMEM_SHARED((ACC_R, 16), jnp.float32),
    ]
    return pl.kernel(
        _sc_deg_body,
        out_type=out_type,
        mesh=mesh,
        scratch_types=scratch,
        compiler_params=pltpu.CompilerParams(use_tc_tiling_on_sc=False),
        name="sc_curv_degree",
    )


R_BLK = 2000
N_BLKS = N_NODES // R_BLK


def _tc_mix_body(a0p, a1p, a0q, a1q, dgp, dgq, Wp, Wn, bp, bn, wv, out):
    degp = jnp.maximum(dgp[:, 0:1], 1.0)
    degn = jnp.maximum(dgq[:, 0:1], 1.0)
    dot = functools.partial(jnp.dot, preferred_element_type=jnp.float32)
    hp = (dot(a0p[...] / degp, Wp[0:D_HALF, :])
          + dot(a1p[...] / degp, Wp[D_HALF:D_FEAT, :]) + bp[0:1, :])
    hn = (dot(a0q[...] / degn, Wn[0:D_HALF, :])
          + dot(a1q[...] / degn, Wn[D_HALF:D_FEAT, :]) + bn[0:1, :])
    out[...] = wv[0, 0] * hp + wv[0, 1] * hn


def _tc_bn_body(h_ref, gamma, beta, ht):
    n = N_NODES
    h = h_ref[...]
    mean = jnp.mean(h, axis=0, keepdims=True)
    var = jnp.mean((h - mean) * (h - mean), axis=0, keepdims=True)
    h = (h - mean) * lax.rsqrt(var + 1e-5) * gamma[0:1, :] + beta[0:1, :]
    h = jnp.maximum(h, 0.0)
    ht[0, :, :] = h[:, 0:D_HALF]
    ht[1, :, :] = h[:, D_HALF:D_FEAT]


def _mk_mix():
    pblk = lambda: pl.BlockSpec((R_BLK, D_HALF), lambda k: (k, 0))
    qblk = lambda: pl.BlockSpec((R_BLK, D_HALF), lambda k: (k + N_BLKS, 0))
    wblk = lambda r, c: pl.BlockSpec((r, c), lambda k: (0, 0))
    return pl.pallas_call(
        _tc_mix_body,
        grid=(N_BLKS,),
        in_specs=[
            pblk(), pblk(), qblk(), qblk(),
            pl.BlockSpec((R_BLK, 16), lambda k: (k, 0)),
            pl.BlockSpec((R_BLK, 16), lambda k: (k + N_BLKS, 0)),
            wblk(D_FEAT, D_FEAT), wblk(D_FEAT, D_FEAT),
            wblk(1, D_FEAT), wblk(1, D_FEAT), wblk(1, 2),
        ],
        out_specs=pl.BlockSpec((R_BLK, D_FEAT), lambda k: (k, 0)),
        out_shape=jax.ShapeDtypeStruct((N_NODES, D_FEAT), jnp.float32),
    )


_tc_mix = _mk_mix()

_tc_bn = pl.pallas_call(
    _tc_bn_body,
    out_shape=jax.ShapeDtypeStruct((2, N_NODES, D_HALF), jnp.float32),
)


def kernel(x, edge_index, edge_curvature, edge_attr, W_pos0, b_pos0, W_neg0,
           b_neg0, W_pos1, b_pos1, W_neg1, b_neg1, cw0, cw1, bn_gamma,
           bn_beta):
    n = N_NODES
    src = edge_index[0].astype(jnp.int32)
    dst = edge_index[1].astype(jnp.int32)
    pos = edge_curvature > 0
    neg = edge_curvature < 0
    dstx = jnp.where(pos, dst, jnp.where(neg, dst + n, DUMMY))
    pad = E_PAD - N_EDGES
    src_p = jnp.concatenate([src, jnp.zeros((pad,), jnp.int32)])
    dstx_p = jnp.concatenate([dstx, jnp.full((pad,), DUMMY, jnp.int32)])
    src2 = jnp.stack([src_p, src_p + n]).reshape(NC, EP_ROWS, CH)
    dstx2 = dstx_p.reshape(EP_ROWS, CH)

    t0 = jnp.concatenate([x[:, :D_HALF], x[:, D_HALF:]], axis=0)

    z64 = jnp.zeros((CH, D_HALF), jnp.float32)
    z16 = jnp.zeros((CH, 16), jnp.float32)
    ones16 = z16.at[:, 0].set(1.0)

    (dg,) = _make_sc_deg()(dstx2, z16, ones16)
    (acc,) = _make_sc_layer()(t0, src2, dstx2, z64)
    dgs = dg[0] + dg[1]

    w0 = jax.nn.softmax(cw0).reshape(1, 2)
    w1 = jax.nn.softmax(cw1).reshape(1, 2)
    r1 = lambda v: v.reshape(1, -1)

    hpre = _tc_mix(acc[0], acc[1], acc[0], acc[1], dgs, dgs, W_pos0, W_neg0,
                   r1(b_pos0), r1(b_neg0), w0)
    ht = _tc_bn(hpre, r1(bn_gamma), r1(bn_beta)).reshape(2 * n, D_HALF)

    (acc1,) = _make_sc_layer()(ht, src2, dstx2, z64)

    out = _tc_mix(acc1[0], acc1[1], acc1[0], acc1[1], dgs, dgs, W_pos1,
                  W_neg1, r1(b_pos1), r1(b_neg1), w1)
    return out
